# baseline (speedup 1.0000x reference)
import math

import jax
import jax.numpy as jnp
from jax import lax
from jax.experimental import pallas as pl
from jax.experimental.pallas import tpu as pltpu
from jax.experimental.pallas import tpu_sc as plsc

D_MODEL = 64
VOCAB = 1000000
BATCH = 4096
SEQ = 200
SCALE = math.sqrt(D_MODEL)

NC, NS, L = 2, 16, 16
NW = NC * NS

PACK_ROWS = VOCAB // 2
ABLK = 256
NBLK_FULL = VOCAB // ABLK
TAIL_V0 = NBLK_FULL * ABLK
A_NBUF = 2
B_NBUF = 4


def _repack_body(tt_hbm, tail_hbm, packed_hbm, *scratch):
    bufs = scratch[:A_NBUF]
    outs = scratch[A_NBUF:2 * A_NBUF]
    isem = scratch[2 * A_NBUF:3 * A_NBUF]
    osem = scratch[3 * A_NBUF:4 * A_NBUF]

    wid = lax.axis_index("s") * NC + lax.axis_index("c")
    nblk_w = (NBLK_FULL - wid + NW - 1) // NW
    lane = lax.iota(jnp.int32, 16)
    diag = [(lane + j) & 15 for j in range(16)]

    @pl.when(wid == 0)
    def _tail():
        pltpu.sync_copy(tail_hbm, packed_hbm.at[pl.ds(TAIL_V0 // 2, 32)])

    def outer(o, carry):
        for b in range(A_NBUF):
            n = o * A_NBUF + b

            @pl.when(n < nblk_w)
            def _start(b=b, n=n):
                @pl.when(o > 0)
                def _drain(b=b):
                    pltpu.make_async_copy(
                        outs[b], packed_hbm.at[pl.ds(0, 128)], osem[b]).wait()

                v0 = pl.multiple_of((wid + n * NW) * ABLK, ABLK)
                pltpu.async_copy(
                    tt_hbm.at[pl.ds(0, D_MODEL), pl.ds(v0, ABLK)],
                    bufs[b], isem[b])

        for b in range(A_NBUF):
            n = o * A_NBUF + b

            @pl.when(n < nblk_w)
            def _work(b=b, n=n):
                v0 = pl.multiple_of((wid + n * NW) * ABLK, ABLK)
                pltpu.make_async_copy(
                    tt_hbm.at[pl.ds(0, D_MODEL), pl.ds(v0, ABLK)],
                    bufs[b], isem[b]).wait()

                for gc in range(8):
                    @plsc.parallel_loop(0, 8)
                    def _pack(h, b=b, gc=gc):
                        mvec = 16 * h + lane
                        scb = 32 * h + 2 * lane + (gc // 4)
                        for j in range(16):
                            cvec = 16 * gc + diag[j]
                            dvec = 16 * (gc & 3) + diag[j]
                            vals = plsc.load_gather(bufs[b], [dvec, scb])
                            plsc.store_scatter(
                                outs[b], [mvec, cvec], vals * SCALE)

                pltpu.async_copy(
                    outs[b],
                    packed_hbm.at[pl.ds(pl.multiple_of(
                        (wid + n * NW) * (ABLK // 2), 8), 128)],
                    osem[b])

        return carry

    a_outer = (NBLK_FULL // NW + 1 + A_NBUF - 1) // A_NBUF
    lax.fori_loop(0, a_outer, outer, 0)
    for b in range(A_NBUF):
        pltpu.make_async_copy(
            outs[b], packed_hbm.at[pl.ds(0, 128)], osem[b]).wait()


_repack = pl.kernel(
    _repack_body,
    out_type=jax.ShapeDtypeStruct((PACK_ROWS, 128), jnp.float32),
    mesh=plsc.VectorSubcoreMesh(
        core_axis_name="c", subcore_axis_name="s",
        num_cores=NC, num_subcores=NS),
    compiler_params=pltpu.CompilerParams(use_tc_tiling_on_sc=True,
                                         needs_layout_passes=False),
    scratch_types=(
        [pltpu.VMEM((D_MODEL, ABLK), jnp.float32) for _ in range(A_NBUF)]
        + [pltpu.VMEM((128, 128), jnp.float32) for _ in range(A_NBUF)]
        + [pltpu.SemaphoreType.DMA for _ in range(2 * A_NBUF)]
    ),
)


def _gather_body(xt_hbm, packed_hbm, out_hbm, *scratch):
    ibufs = scratch[:B_NBUF]
    jbufs = scratch[B_NBUF:2 * B_NBUF]
    rows = scratch[2 * B_NBUF:3 * B_NBUF]
    slabs = scratch[3 * B_NBUF:4 * B_NBUF]
    isem = scratch[4 * B_NBUF:5 * B_NBUF]
    gsem = scratch[5 * B_NBUF:6 * B_NBUF]
    osem = scratch[6 * B_NBUF:7 * B_NBUF]

    wid = lax.axis_index("s") * NC + lax.axis_index("c")
    b0 = pl.multiple_of(wid * 128, 128)
    lane = lax.iota(jnp.int32, 16)
    diag = [(lane + j) & 15 for j in range(16)]

    def outer(o, carry):
        for b in range(B_NBUF):
            t = o * B_NBUF + b
            pltpu.async_copy(
                xt_hbm.at[t, pl.ds(b0, 128)], ibufs[b], isem[b])

        for b in range(B_NBUF):
            t = o * B_NBUF + b
            pltpu.make_async_copy(
                xt_hbm.at[t, pl.ds(b0, 128)], ibufs[b], isem[b]).wait()

            def shift_row(k, c2, b=b):
                jbufs[b][pl.ds(16 * k, 16)] = lax.shift_right_logical(
                    ibufs[b][pl.ds(16 * k, 16)], 1)
                return c2

            lax.fori_loop(0, 8, shift_row, 0)
            pltpu.async_copy(packed_hbm.at[jbufs[b]], rows[b], gsem[b])

        for b in range(B_NBUF):
            t = o * B_NBUF + b

            @pl.when(o > 0)
            def _drain(b=b):
                pltpu.make_async_copy(
                    slabs[b], out_hbm.at[0, pl.ds(0, D_MODEL), pl.ds(0, 128)],
                    osem[b]).wait()

            pltpu.make_async_copy(
                packed_hbm.at[jbufs[b]], rows[b], gsem[b]).wait()

            for g in range(4):
                @plsc.parallel_loop(0, 8)
                def _trans(h, b=b, g=g):
                    ivec = 16 * h + lane
                    iv = ibufs[b][pl.ds(16 * h, 16)]
                    par64 = (iv & 1) * 64 + 16 * g
                    for j in range(16):
                        vals = plsc.load_gather(
                            rows[b], [ivec, par64 + diag[j]])
                        plsc.store_scatter(
                            slabs[b], [16 * g + diag[j], ivec], vals)

            pltpu.async_copy(
                slabs[b],
                out_hbm.at[t, pl.ds(0, D_MODEL), pl.ds(b0, 128)], osem[b])
        return carry

    lax.fori_loop(0, SEQ // B_NBUF, outer, 0)
    for b in range(B_NBUF):
        pltpu.make_async_copy(
            slabs[b], out_hbm.at[0, pl.ds(0, D_MODEL), pl.ds(0, 128)],
            osem[b]).wait()


_gather = pl.kernel(
    _gather_body,
    out_type=jax.ShapeDtypeStruct((SEQ, D_MODEL, BATCH), jnp.float32),
    mesh=plsc.VectorSubcoreMesh(
        core_axis_name="c", subcore_axis_name="s",
        num_cores=NC, num_subcores=NS),
    compiler_params=pltpu.CompilerParams(use_tc_tiling_on_sc=True,
                                         needs_layout_passes=False),
    scratch_types=(
        [pltpu.VMEM((128,), jnp.int32) for _ in range(B_NBUF)]
        + [pltpu.VMEM((128,), jnp.int32) for _ in range(B_NBUF)]
        + [pltpu.VMEM((128, 128), jnp.float32) for _ in range(B_NBUF)]
        + [pltpu.VMEM((D_MODEL, 128), jnp.float32) for _ in range(B_NBUF)]
        + [pltpu.SemaphoreType.DMA for _ in range(3 * B_NBUF)]
    ),
)


def kernel(x, table):
    tail = (table[TAIL_V0:] * SCALE).reshape(32, 128)
    packed = _repack(table.T, tail)
    out_phys = _gather(x.T, packed)
    return out_phys.transpose(2, 0, 1)


# speedup vs baseline: 1.0748x; 1.0748x over previous
import math

import jax
import jax.numpy as jnp
from jax import lax
from jax.experimental import pallas as pl
from jax.experimental.pallas import tpu as pltpu
from jax.experimental.pallas import tpu_sc as plsc

D_MODEL = 64
VOCAB = 1000000
BATCH = 4096
SEQ = 200
SCALE = math.sqrt(D_MODEL)

NC, NS, L = 2, 16, 16
NW = NC * NS

PACK_ROWS = VOCAB // 2
ABLK = 256
NBLK_FULL = VOCAB // ABLK
TAIL_V0 = NBLK_FULL * ABLK
A_NBUF = 2
B_NBUF = 4


def _repack_body(tt_hbm, tail_hbm, packed_hbm, *scratch):
    bufs = scratch[:A_NBUF]
    outs = scratch[A_NBUF:2 * A_NBUF]
    isem = scratch[2 * A_NBUF:3 * A_NBUF]
    osem = scratch[3 * A_NBUF:4 * A_NBUF]

    wid = lax.axis_index("s") * NC + lax.axis_index("c")
    nblk_w = (NBLK_FULL - wid + NW - 1) // NW
    lane = lax.iota(jnp.int32, 16)
    diag = [(lane + j) & 15 for j in range(16)]

    @pl.when(wid == 0)
    def _tail():
        pltpu.sync_copy(tail_hbm, packed_hbm.at[pl.ds(TAIL_V0 // 2, 32)])

    def outer(o, carry):
        for b in range(A_NBUF):
            n = o * A_NBUF + b

            @pl.when(n < nblk_w)
            def _start(b=b, n=n):
                @pl.when(o > 0)
                def _drain(b=b):
                    pltpu.make_async_copy(
                        outs[b], packed_hbm.at[pl.ds(0, 128)], osem[b]).wait()

                v0 = pl.multiple_of((wid + n * NW) * ABLK, ABLK)
                pltpu.async_copy(
                    tt_hbm.at[pl.ds(0, D_MODEL), pl.ds(v0, ABLK)],
                    bufs[b], isem[b])

        for b in range(A_NBUF):
            n = o * A_NBUF + b

            @pl.when(n < nblk_w)
            def _work(b=b, n=n):
                v0 = pl.multiple_of((wid + n * NW) * ABLK, ABLK)
                pltpu.make_async_copy(
                    tt_hbm.at[pl.ds(0, D_MODEL), pl.ds(v0, ABLK)],
                    bufs[b], isem[b]).wait()

                for gc in range(8):
                    @plsc.parallel_loop(0, 8)
                    def _pack(h, b=b, gc=gc):
                        mvec = 16 * h + lane
                        scb = 32 * h + 2 * lane + (gc // 4)
                        for j in range(16):
                            cvec = 16 * gc + diag[j]
                            dvec = 16 * (gc & 3) + diag[j]
                            vals = plsc.load_gather(bufs[b], [dvec, scb])
                            plsc.store_scatter(
                                outs[b], [mvec, cvec], vals * SCALE)

                pltpu.async_copy(
                    outs[b],
                    packed_hbm.at[pl.ds(pl.multiple_of(
                        (wid + n * NW) * (ABLK // 2), 8), 128)],
                    osem[b])

        return carry

    a_outer = (NBLK_FULL // NW + 1 + A_NBUF - 1) // A_NBUF
    lax.fori_loop(0, a_outer, outer, 0)
    for b in range(A_NBUF):
        pltpu.make_async_copy(
            outs[b], packed_hbm.at[pl.ds(0, 128)], osem[b]).wait()


_repack = pl.kernel(
    _repack_body,
    out_type=jax.ShapeDtypeStruct((PACK_ROWS, 128), jnp.float32),
    mesh=plsc.VectorSubcoreMesh(
        core_axis_name="c", subcore_axis_name="s",
        num_cores=NC, num_subcores=NS),
    compiler_params=pltpu.CompilerParams(use_tc_tiling_on_sc=True,
                                         needs_layout_passes=False),
    scratch_types=(
        [pltpu.VMEM((D_MODEL, ABLK), jnp.float32) for _ in range(A_NBUF)]
        + [pltpu.VMEM((128, 128), jnp.float32) for _ in range(A_NBUF)]
        + [pltpu.SemaphoreType.DMA for _ in range(2 * A_NBUF)]
    ),
)


def _gather_body(xt_hbm, packed_hbm, out_hbm, *scratch):
    ibufs = scratch[:B_NBUF]
    jbufs = scratch[B_NBUF:2 * B_NBUF]
    rows = scratch[2 * B_NBUF:3 * B_NBUF]
    slabs = scratch[3 * B_NBUF:4 * B_NBUF]
    isem = scratch[4 * B_NBUF:5 * B_NBUF]
    gsem = scratch[5 * B_NBUF:6 * B_NBUF]
    osem = scratch[6 * B_NBUF:7 * B_NBUF]

    wid = lax.axis_index("s") * NC + lax.axis_index("c")
    b0 = pl.multiple_of(wid * 128, 128)
    lane = lax.iota(jnp.int32, 16)
    diag = [(lane + j) & 15 for j in range(16)]

    def outer(o, carry):
        for b in range(B_NBUF):
            t = o * B_NBUF + b
            pltpu.async_copy(
                xt_hbm.at[t, pl.ds(b0, 128)], ibufs[b], isem[b])

        for b in range(B_NBUF):
            t = o * B_NBUF + b
            pltpu.make_async_copy(
                xt_hbm.at[t, pl.ds(b0, 128)], ibufs[b], isem[b]).wait()

            def shift_row(k, c2, b=b):
                jbufs[b][pl.ds(16 * k, 16)] = lax.shift_right_logical(
                    ibufs[b][pl.ds(16 * k, 16)], 1)
                return c2

            lax.fori_loop(0, 8, shift_row, 0)
            pltpu.async_copy(packed_hbm.at[jbufs[b]], rows[b], gsem[b])

        for b in range(B_NBUF):
            t = o * B_NBUF + b

            @pl.when(o > 0)
            def _drain(b=b):
                pltpu.make_async_copy(
                    slabs[b], out_hbm.at[0, pl.ds(0, D_MODEL), pl.ds(0, 128)],
                    osem[b]).wait()

            pltpu.make_async_copy(
                packed_hbm.at[jbufs[b]], rows[b], gsem[b]).wait()

            for g in range(4):
                @plsc.parallel_loop(0, 8)
                def _trans(h, b=b, g=g):
                    ivec = 16 * h + lane
                    iv = ibufs[b][pl.ds(16 * h, 16)]
                    par64 = (iv & 1) * 64 + 16 * g
                    for j in range(16):
                        vals = plsc.load_gather(
                            rows[b], [ivec, par64 + diag[j]])
                        plsc.store_scatter(
                            slabs[b], [16 * g + diag[j], ivec], vals * SCALE)

            pltpu.async_copy(
                slabs[b],
                out_hbm.at[t, pl.ds(0, D_MODEL), pl.ds(b0, 128)], osem[b])
        return carry

    lax.fori_loop(0, SEQ // B_NBUF, outer, 0)
    for b in range(B_NBUF):
        pltpu.make_async_copy(
            slabs[b], out_hbm.at[0, pl.ds(0, D_MODEL), pl.ds(0, 128)],
            osem[b]).wait()


_gather = pl.kernel(
    _gather_body,
    out_type=jax.ShapeDtypeStruct((SEQ, D_MODEL, BATCH), jnp.float32),
    mesh=plsc.VectorSubcoreMesh(
        core_axis_name="c", subcore_axis_name="s",
        num_cores=NC, num_subcores=NS),
    compiler_params=pltpu.CompilerParams(use_tc_tiling_on_sc=True,
                                         needs_layout_passes=False),
    scratch_types=(
        [pltpu.VMEM((128,), jnp.int32) for _ in range(B_NBUF)]
        + [pltpu.VMEM((128,), jnp.int32) for _ in range(B_NBUF)]
        + [pltpu.VMEM((128, 128), jnp.float32) for _ in range(B_NBUF)]
        + [pltpu.VMEM((D_MODEL, 128), jnp.float32) for _ in range(B_NBUF)]
        + [pltpu.SemaphoreType.DMA for _ in range(3 * B_NBUF)]
    ),
)


def kernel(x, table):
    packed = table.reshape(PACK_ROWS, 128)
    out_phys = _gather(x.T, packed)
    return out_phys.transpose(2, 0, 1)


# speedup vs baseline: 1.1004x; 1.0238x over previous
import math

import jax
import jax.numpy as jnp
from jax import lax
from jax.experimental import pallas as pl
from jax.experimental.pallas import tpu as pltpu
from jax.experimental.pallas import tpu_sc as plsc

D_MODEL = 64
VOCAB = 1000000
BATCH = 4096
SEQ = 200
SCALE = math.sqrt(D_MODEL)

NC, NS, L = 2, 16, 16
NW = NC * NS
ROWS_PER_W = BATCH // NW
NBUF = 8
OUTER = ROWS_PER_W // NBUF


def _emb_body(x_hbm, table_hbm, out_hbm, idx_v, *scratch):
    rows = scratch[:NBUF]
    gsem = scratch[NBUF:2 * NBUF]
    ssem = scratch[2 * NBUF:3 * NBUF]

    wid = lax.axis_index("s") * NC + lax.axis_index("c")
    row0 = wid * ROWS_PER_W

    pltpu.sync_copy(x_hbm.at[pl.ds(row0, ROWS_PER_W)], idx_v)

    def outer(o, carry):
        for b in range(NBUF):
            i = o * NBUF + b

            @pl.when(o > 0)
            def _drain_prev_scatter(b=b):
                pltpu.make_async_copy(
                    rows[b], out_hbm.at[0], ssem[b]).wait()

            pltpu.async_copy(table_hbm.at[idx_v.at[i]], rows[b], gsem[b])
        for b in range(NBUF):
            i = o * NBUF + b
            pltpu.make_async_copy(
                table_hbm.at[idx_v.at[i]], rows[b], gsem[b]).wait()

            def scale_row(t, c, b=b):
                for j in range(D_MODEL // L):
                    rows[b][t, pl.ds(j * L, L)] = (
                        rows[b][t, pl.ds(j * L, L)] * SCALE)
                return c

            lax.fori_loop(0, SEQ, scale_row, 0)
            pltpu.async_copy(rows[b], out_hbm.at[row0 + i], ssem[b])
        return carry

    lax.fori_loop(0, OUTER, outer, 0)
    for b in range(NBUF):
        pltpu.make_async_copy(rows[b], out_hbm.at[0], ssem[b]).wait()


_emb = pl.kernel(
    _emb_body,
    out_type=jax.ShapeDtypeStruct((BATCH, SEQ, D_MODEL), jnp.float32),
    mesh=plsc.VectorSubcoreMesh(
        core_axis_name="c", subcore_axis_name="s",
        num_cores=NC, num_subcores=NS),
    compiler_params=pltpu.CompilerParams(use_tc_tiling_on_sc=False),
    scratch_types=(
        [pltpu.VMEM((ROWS_PER_W, SEQ), jnp.int32)]
        + [pltpu.VMEM((SEQ, D_MODEL), jnp.float32) for _ in range(NBUF)]
        + [pltpu.SemaphoreType.DMA for _ in range(2 * NBUF)]
    ),
)


def kernel(x, table):
    return _emb(x, table)
